# baseline (speedup 1.0000x reference)
import functools
import jax
import jax.numpy as jnp
from jax import lax
from jax.experimental import pallas as pl
from jax.experimental.pallas import tpu as pltpu
from jax.experimental.pallas import tpu_sc as plsc

N = 10000
E = 320000
D_H = 128
D_OUT = 64
BETA = 0.1

NC = 2
NS = 16
NW = NC * NS
CHUNK = 104
FULL = 96
ECHUNK = 64
EXTRA_BASE = NW * FULL * CHUNK
EPT = E // NW
NPAD = 10240
STRIPE = NPAD // NS

_sc_params = pltpu.CompilerParams(needs_layout_passes=False)


@functools.cache
def _sc_mesh():
    return plsc.VectorSubcoreMesh(core_axis_name="c", subcore_axis_name="s",
                                  num_cores=NC, num_subcores=NS)


def _deg_body(dst_hbm, out_hbm, dstv, hist):
    cid = lax.axis_index("c")
    sid = lax.axis_index("s")
    wid = sid * NC + cid

    z16 = jnp.zeros((16,), jnp.float32)

    def zero_row(i, _):
        hist[pl.ds(i * 16, 16)] = z16
        return 0

    lax.fori_loop(0, NPAD // 16, zero_row, 0)

    pltpu.sync_copy(dst_hbm.at[pl.ds(wid * EPT, EPT)], dstv)

    ones = jnp.full((16,), 1.0, jnp.float32)

    def count(i, _):
        d = dstv[pl.ds(i * 16, 16)]
        plsc.addupdate_scatter(hist, [d], ones)
        return 0

    lax.fori_loop(0, EPT // 16, count, 0)
    pltpu.sync_copy(hist, out_hbm.at[wid])


def _deg_partials(dst):
    return pl.kernel(
        _deg_body,
        out_type=jax.ShapeDtypeStruct((NW, NPAD), jnp.float32),
        mesh=_sc_mesh(),
        scratch_types=[
            pltpu.VMEM((EPT,), jnp.int32),
            pltpu.VMEM((NPAD,), jnp.float32),
        ],
        compiler_params=_sc_params,
    )(dst)


def _scatter_body(y_hbm, src_hbm, dst_hbm, out_hbm,
                  ssm, dsm, rows, dsm_e, isem, gsem, acc):
    cid = lax.axis_index("c")
    sid = lax.axis_index("s")
    wid = sid * NC + cid
    base = wid * (FULL * CHUNK)

    def idx_start(c, ib):
        off = base + c * CHUNK
        pltpu.async_copy(src_hbm.at[pl.ds(off, CHUNK)], ssm[ib], isem[ib])
        pltpu.async_copy(dst_hbm.at[pl.ds(off, CHUNK)], dsm[ib], isem[ib])

    def idx_wait(c, ib):
        off = base + c * CHUNK
        pltpu.make_async_copy(src_hbm.at[pl.ds(off, CHUNK)], ssm[ib],
                              isem[ib]).wait()
        pltpu.make_async_copy(dst_hbm.at[pl.ds(off, CHUNK)], dsm[ib],
                              isem[ib]).wait()

    def gather_start(ib, rb):
        pltpu.async_copy(y_hbm.at[ssm[ib]], rows[rb], gsem[rb])

    def gather_wait(ib, rb):
        pltpu.make_async_copy(y_hbm.at[ssm[ib]], rows[rb], gsem[rb]).wait()

    idx_start(0, 0)
    idx_start(1, 1)
    idx_start(2, 2)

    z16 = jnp.zeros((16,), jnp.float32)

    def zero_row(i, _):
        for t in range(D_H // 16):
            rows[0][i, pl.ds(t * 16, 16)] = z16
        return 0

    lax.fori_loop(0, CHUNK, zero_row, 0)
    for k in range(STRIPE // CHUNK + 1):
        w = min(CHUNK, STRIPE - k * CHUNK)
        if w > 0:
            pltpu.sync_copy(rows[0].at[pl.ds(0, w)],
                            acc.at[pl.ds(sid * STRIPE + k * CHUNK, w)])

    idx_wait(0, 0)
    gather_start(0, 0)
    idx_wait(1, 1)
    gather_start(1, 1)
    plsc.subcore_barrier()

    def body(j, _):
        c0 = 6 * j
        for b in range(6):
            c = c0 + b
            ib, ib2 = b % 3, (b + 2) % 3

            gather_wait(ib, ib)

            @pl.when(c + 2 < FULL)
            def _():
                idx_wait(c + 2, ib2)
                gather_start(ib2, ib2)

            pltpu.sync_copy(rows[ib], acc.at[dsm[ib]], add=True)

            @pl.when(c + 3 < FULL)
            def _():
                idx_start(c + 3, ib)
        return 0

    lax.fori_loop(0, FULL // 6, body, 0)

    @pl.when(wid < 8)
    def _():
        ebase = EXTRA_BASE + wid * ECHUNK
        pltpu.sync_copy(src_hbm.at[pl.ds(ebase, ECHUNK)],
                        ssm[0].at[pl.ds(0, ECHUNK)])
        pltpu.async_copy(y_hbm.at[ssm[0].at[pl.ds(0, ECHUNK)]],
                         rows[0].at[pl.ds(0, ECHUNK)], gsem[0]).wait()
        pltpu.sync_copy(dst_hbm.at[pl.ds(ebase, ECHUNK)], dsm_e)
        pltpu.sync_copy(rows[0].at[pl.ds(0, ECHUNK)],
                        acc.at[dsm_e], add=True)

    plsc.subcore_barrier()
    pltpu.sync_copy(acc.at[pl.ds(sid * STRIPE, STRIPE)],
                    out_hbm.at[cid, pl.ds(sid * STRIPE, STRIPE)])


def _edge_scatter(y, src, dst):
    return pl.kernel(
        _scatter_body,
        out_type=jax.ShapeDtypeStruct((NC, NPAD, D_H), jnp.float32),
        mesh=_sc_mesh(),
        scratch_types=[
            [pltpu.VMEM((CHUNK,), jnp.int32) for _ in range(3)],
            [pltpu.VMEM((CHUNK,), jnp.int32) for _ in range(3)],
            [pltpu.VMEM((CHUNK, D_H), jnp.float32) for _ in range(3)],
            pltpu.VMEM((ECHUNK,), jnp.int32),
            [pltpu.SemaphoreType.DMA for _ in range(3)],
            [pltpu.SemaphoreType.DMA for _ in range(3)],
            pltpu.VMEM_SHARED((NPAD, D_H), jnp.float32),
        ],
        compiler_params=_sc_params,
    )(y, src, dst)


BLK = 1024
GRID = NPAD // BLK


def _norm_accum(h, hn_ref, hth_ref):
    nrm = jnp.sqrt(jnp.sum(h * h, axis=1, keepdims=True))
    hn = h / (nrm + 1e-12)
    hn_ref[...] = hn
    hth = lax.dot_general(hn, hn, (((0,), (0,)), ((), ())),
                          preferred_element_type=jnp.float32)

    @pl.when(pl.program_id(0) == 0)
    def _():
        hth_ref[...] = jnp.zeros_like(hth_ref)

    hth_ref[...] += hth


def _y1_body(degp_ref, h_ref, w_ref, dinv_ref, y_ref, hn_ref, hth_ref):
    deg = jnp.sum(degp_ref[...], axis=0, keepdims=True) + 1.0
    dinv = lax.rsqrt(deg).T
    dinv_ref[...] = dinv
    h = h_ref[...]
    y_ref[...] = jnp.dot(h, w_ref[...],
                         preferred_element_type=jnp.float32) * dinv
    _norm_accum(h, hn_ref, hth_ref)


def _layer_y1(degp, h, W):
    return pl.pallas_call(
        _y1_body,
        grid=(GRID,),
        in_specs=[
            pl.BlockSpec((NW, BLK), lambda i: (0, i)),
            pl.BlockSpec((BLK, D_H), lambda i: (i, 0)),
            pl.BlockSpec((D_H, D_H), lambda i: (0, 0)),
        ],
        out_specs=[
            pl.BlockSpec((BLK, 1), lambda i: (i, 0)),
            pl.BlockSpec((BLK, D_H), lambda i: (i, 0)),
            pl.BlockSpec((BLK, D_H), lambda i: (i, 0)),
            pl.BlockSpec((D_H, D_H), lambda i: (0, 0)),
        ],
        out_shape=[
            jax.ShapeDtypeStruct((NPAD, 1), jnp.float32),
            jax.ShapeDtypeStruct((NPAD, D_H), jnp.float32),
            jax.ShapeDtypeStruct((NPAD, D_H), jnp.float32),
            jax.ShapeDtypeStruct((D_H, D_H), jnp.float32),
        ],
    )(degp, h, W)


def _post_h(s0, s1, y, dinv, hn, hth, wv, b, mask):
    m = jnp.dot(hth[...], wv[...],
                preferred_element_type=jnp.float32) * (BETA / N)
    z = ((s0[...] + s1[...] + y[...]) * dinv[...]
         + jnp.dot(hn[...], m, preferred_element_type=jnp.float32)
         + b[...])
    z = jnp.maximum(z, 0.0)
    nrm = jnp.sqrt(jnp.sum(z * z, axis=1, keepdims=True))
    return (z / (nrm + 1e-12)) * mask[...]


def _post_mid_body(s0, s1, y, dinv, hn, hth, wv, b, mask, w2,
                   y2_ref, hn2_ref, hth2_ref):
    h = _post_h(s0, s1, y, dinv, hn, hth, wv, b, mask)
    y2_ref[...] = jnp.dot(h, w2[...],
                          preferred_element_type=jnp.float32) * dinv[...]
    _norm_accum(h, hn2_ref, hth2_ref)


def _post_final_body(s0, s1, y, dinv, hn, hth, wv, b, mask, wc, bc, o_ref):
    h = _post_h(s0, s1, y, dinv, hn, hth, wv, b, mask)
    o_ref[...] = jnp.dot(h, wc[...],
                         preferred_element_type=jnp.float32) + bc[...]


def _layer_post(Sp, y, dinv, hn, hth, WV, b, mask, final=None, mid=None):
    row = lambda: pl.BlockSpec((BLK, D_H), lambda i: (i, 0))
    sq = pl.BlockSpec((D_H, D_H), lambda i: (0, 0))
    col = pl.BlockSpec((BLK, 1), lambda i: (i, 0))
    in_specs = [row(), row(), row(), col, row(), sq, sq,
                pl.BlockSpec((1, D_H), lambda i: (0, 0)), col]
    args = [Sp[0], Sp[1], y, dinv, hn, hth, WV, b.reshape(1, D_H), mask]
    if mid is not None:
        body = _post_mid_body
        in_specs += [sq]
        args += [mid]
        out_spec = [row(), row(), sq]
        out_shape = [jax.ShapeDtypeStruct((NPAD, D_H), jnp.float32),
                     jax.ShapeDtypeStruct((NPAD, D_H), jnp.float32),
                     jax.ShapeDtypeStruct((D_H, D_H), jnp.float32)]
    else:
        Wc, bc = final
        body = _post_final_body
        in_specs += [pl.BlockSpec((D_H, D_OUT), lambda i: (0, 0)),
                     pl.BlockSpec((1, D_OUT), lambda i: (0, 0))]
        args += [Wc, bc.reshape(1, D_OUT)]
        out_spec = pl.BlockSpec((BLK, D_OUT), lambda i: (i, 0))
        out_shape = jax.ShapeDtypeStruct((NPAD, D_OUT), jnp.float32)
    return pl.pallas_call(
        body,
        grid=(GRID,),
        in_specs=in_specs,
        out_specs=out_spec,
        out_shape=out_shape,
    )(*args)


def kernel(x, edge_index, W1, b1, WV1, W2, b2, WV2, Wc, bc):
    src = edge_index[0].astype(jnp.int32)
    dst = edge_index[1].astype(jnp.int32)

    degp = _deg_partials(dst)
    mask = (jnp.arange(NPAD) < N).astype(jnp.float32)[:, None]

    h = jnp.zeros((NPAD, D_H), jnp.float32).at[:N].set(x)

    dinv, y, hn, hth = _layer_y1(degp, h, W1)
    Sp = _edge_scatter(y, src, dst)
    y, hn, hth = _layer_post(Sp, y, dinv, hn, hth, WV1, b1, mask, mid=W2)

    Sp = _edge_scatter(y, src, dst)
    outp = _layer_post(Sp, y, dinv, hn, hth, WV2, b2, mask, final=(Wc, bc))

    out = outp[:N]
    qel = jnp.array(0.0, dtype=jnp.float32)
    return (out, qel)


# speedup vs baseline: 1.1208x; 1.1208x over previous
import functools
import jax
import jax.numpy as jnp
from jax import lax
from jax.experimental import pallas as pl
from jax.experimental.pallas import tpu as pltpu
from jax.experimental.pallas import tpu_sc as plsc

N = 10000
E = 320000
D_H = 128
D_OUT = 64
BETA = 0.1

NC = 2
NS = 16
NW = NC * NS
CHUNK = 128
FULL = 78
EXTRA_BASE = NW * FULL * CHUNK
EPT = E // NW
NPAD = 10240
STRIPE = NPAD // NS

_sc_params = pltpu.CompilerParams(needs_layout_passes=False)


@functools.cache
def _sc_mesh():
    return plsc.VectorSubcoreMesh(core_axis_name="c", subcore_axis_name="s",
                                  num_cores=NC, num_subcores=NS)


def _deg_body(dst_hbm, out_hbm, dstv, hist):
    cid = lax.axis_index("c")
    sid = lax.axis_index("s")
    wid = sid * NC + cid

    z16 = jnp.zeros((16,), jnp.float32)

    def zero_row(i, _):
        hist[pl.ds(i * 16, 16)] = z16
        return 0

    lax.fori_loop(0, NPAD // 16, zero_row, 0)

    pltpu.sync_copy(dst_hbm.at[pl.ds(wid * EPT, EPT)], dstv)

    ones = jnp.full((16,), 1.0, jnp.float32)

    def count(i, _):
        d = dstv[pl.ds(i * 16, 16)]
        plsc.addupdate_scatter(hist, [d], ones)
        return 0

    lax.fori_loop(0, EPT // 16, count, 0)
    pltpu.sync_copy(hist, out_hbm.at[wid])


def _deg_partials(dst):
    return pl.kernel(
        _deg_body,
        out_type=jax.ShapeDtypeStruct((NW, NPAD), jnp.float32),
        mesh=_sc_mesh(),
        scratch_types=[
            pltpu.VMEM((EPT,), jnp.int32),
            pltpu.VMEM((NPAD,), jnp.float32),
        ],
        compiler_params=_sc_params,
    )(dst)


def _scatter_body(y_hbm, src_hbm, dst_hbm, out_hbm,
                  ssm, dsm, rows, isem, gsem, acc):
    cid = lax.axis_index("c")
    sid = lax.axis_index("s")
    wid = sid * NC + cid
    base = wid * (FULL * CHUNK)

    def idx_start(c, ib):
        off = base + c * CHUNK
        pltpu.async_copy(src_hbm.at[pl.ds(off, CHUNK)], ssm[ib], isem[ib])
        pltpu.async_copy(dst_hbm.at[pl.ds(off, CHUNK)], dsm[ib], isem[ib])

    def idx_wait(c, ib):
        off = base + c * CHUNK
        pltpu.make_async_copy(src_hbm.at[pl.ds(off, CHUNK)], ssm[ib],
                              isem[ib]).wait()
        pltpu.make_async_copy(dst_hbm.at[pl.ds(off, CHUNK)], dsm[ib],
                              isem[ib]).wait()

    def gather_start(ib, rb):
        pltpu.async_copy(y_hbm.at[ssm[ib]], rows[rb], gsem[rb])

    def gather_wait(ib, rb):
        pltpu.make_async_copy(y_hbm.at[ssm[ib]], rows[rb], gsem[rb]).wait()

    idx_start(0, 0)
    idx_start(1, 1)

    z16 = jnp.zeros((16,), jnp.float32)

    def zero_row(i, _):
        for t in range(D_H // 16):
            rows[0][i, pl.ds(t * 16, 16)] = z16
        return 0

    lax.fori_loop(0, CHUNK, zero_row, 0)
    for k in range(STRIPE // CHUNK):
        pltpu.sync_copy(rows[0], acc.at[pl.ds(sid * STRIPE + k * CHUNK,
                                              CHUNK)])

    idx_wait(0, 0)
    gather_start(0, 0)
    plsc.subcore_barrier()

    def body(j, _):
        c0 = 6 * j
        for b in range(6):
            c = c0 + b
            ib, ib1, ib2 = b % 3, (b + 1) % 3, (b + 2) % 3
            rb, rb1 = b % 2, (b + 1) % 2

            @pl.when(c + 2 < FULL)
            def _():
                idx_start(c + 2, ib2)

            @pl.when(c + 1 < FULL)
            def _():
                idx_wait(c + 1, ib1)
                gather_start(ib1, rb1)

            gather_wait(ib, rb)
            pltpu.sync_copy(rows[rb], acc.at[dsm[ib]], add=True)
        return 0

    lax.fori_loop(0, FULL // 6, body, 0)

    @pl.when(wid < 4)
    def _():
        ebase = EXTRA_BASE + wid * CHUNK
        pltpu.sync_copy(src_hbm.at[pl.ds(ebase, CHUNK)], ssm[0])
        pltpu.async_copy(y_hbm.at[ssm[0]], rows[0], gsem[0]).wait()
        pltpu.sync_copy(dst_hbm.at[pl.ds(ebase, CHUNK)], dsm[0])
        pltpu.sync_copy(rows[0], acc.at[dsm[0]], add=True)

    plsc.subcore_barrier()
    pltpu.sync_copy(acc.at[pl.ds(sid * STRIPE, STRIPE)],
                    out_hbm.at[cid, pl.ds(sid * STRIPE, STRIPE)])


def _edge_scatter(y, src, dst):
    return pl.kernel(
        _scatter_body,
        out_type=jax.ShapeDtypeStruct((NC, NPAD, D_H), jnp.float32),
        mesh=_sc_mesh(),
        scratch_types=[
            [pltpu.VMEM((CHUNK,), jnp.int32) for _ in range(3)],
            [pltpu.VMEM((CHUNK,), jnp.int32) for _ in range(3)],
            [pltpu.VMEM((CHUNK, D_H), jnp.float32) for _ in range(2)],
            [pltpu.SemaphoreType.DMA for _ in range(3)],
            [pltpu.SemaphoreType.DMA for _ in range(2)],
            pltpu.VMEM_SHARED((NPAD, D_H), jnp.float32),
        ],
        compiler_params=_sc_params,
    )(y, src, dst)


BLK = 2048
GRID = NPAD // BLK


def _norm_accum(h, hn_ref, hth_ref):
    nrm = jnp.sqrt(jnp.sum(h * h, axis=1, keepdims=True))
    hn = h / (nrm + 1e-12)
    hn_ref[...] = hn
    hth = lax.dot_general(hn, hn, (((0,), (0,)), ((), ())),
                          preferred_element_type=jnp.float32)

    @pl.when(pl.program_id(0) == 0)
    def _():
        hth_ref[...] = jnp.zeros_like(hth_ref)

    hth_ref[...] += hth


def _y1_body(degp_ref, h_ref, w_ref, dinv_ref, y_ref, hn_ref, hth_ref):
    deg = jnp.sum(degp_ref[...], axis=0, keepdims=True) + 1.0
    dinv = lax.rsqrt(deg).T
    dinv_ref[...] = dinv
    h = h_ref[...]
    y_ref[...] = jnp.dot(h, w_ref[...],
                         preferred_element_type=jnp.float32) * dinv
    _norm_accum(h, hn_ref, hth_ref)


def _layer_y1(degp, h, W):
    return pl.pallas_call(
        _y1_body,
        grid=(GRID,),
        in_specs=[
            pl.BlockSpec((NW, BLK), lambda i: (0, i)),
            pl.BlockSpec((BLK, D_H), lambda i: (i, 0)),
            pl.BlockSpec((D_H, D_H), lambda i: (0, 0)),
        ],
        out_specs=[
            pl.BlockSpec((BLK, 1), lambda i: (i, 0)),
            pl.BlockSpec((BLK, D_H), lambda i: (i, 0)),
            pl.BlockSpec((BLK, D_H), lambda i: (i, 0)),
            pl.BlockSpec((D_H, D_H), lambda i: (0, 0)),
        ],
        out_shape=[
            jax.ShapeDtypeStruct((NPAD, 1), jnp.float32),
            jax.ShapeDtypeStruct((NPAD, D_H), jnp.float32),
            jax.ShapeDtypeStruct((NPAD, D_H), jnp.float32),
            jax.ShapeDtypeStruct((D_H, D_H), jnp.float32),
        ],
    )(degp, h, W)


def _post_h(s0, s1, y, dinv, hn, hth, wv, b, mask):
    m = jnp.dot(hth[...], wv[...],
                preferred_element_type=jnp.float32) * (BETA / N)
    z = ((s0[...] + s1[...] + y[...]) * dinv[...]
         + jnp.dot(hn[...], m, preferred_element_type=jnp.float32)
         + b[...])
    z = jnp.maximum(z, 0.0)
    nrm = jnp.sqrt(jnp.sum(z * z, axis=1, keepdims=True))
    return (z / (nrm + 1e-12)) * mask[...]


def _post_mid_body(s0, s1, y, dinv, hn, hth, wv, b, mask, w2,
                   y2_ref, hn2_ref, hth2_ref):
    h = _post_h(s0, s1, y, dinv, hn, hth, wv, b, mask)
    y2_ref[...] = jnp.dot(h, w2[...],
                          preferred_element_type=jnp.float32) * dinv[...]
    _norm_accum(h, hn2_ref, hth2_ref)


def _post_final_body(s0, s1, y, dinv, hn, hth, wv, b, mask, wc, bc, o_ref):
    h = _post_h(s0, s1, y, dinv, hn, hth, wv, b, mask)
    o_ref[...] = jnp.dot(h, wc[...],
                         preferred_element_type=jnp.float32) + bc[...]


def _layer_post(Sp, y, dinv, hn, hth, WV, b, mask, final=None, mid=None):
    row = lambda: pl.BlockSpec((BLK, D_H), lambda i: (i, 0))
    sq = pl.BlockSpec((D_H, D_H), lambda i: (0, 0))
    col = pl.BlockSpec((BLK, 1), lambda i: (i, 0))
    in_specs = [row(), row(), row(), col, row(), sq, sq,
                pl.BlockSpec((1, D_H), lambda i: (0, 0)), col]
    args = [Sp[0], Sp[1], y, dinv, hn, hth, WV, b.reshape(1, D_H), mask]
    if mid is not None:
        body = _post_mid_body
        in_specs += [sq]
        args += [mid]
        out_spec = [row(), row(), sq]
        out_shape = [jax.ShapeDtypeStruct((NPAD, D_H), jnp.float32),
                     jax.ShapeDtypeStruct((NPAD, D_H), jnp.float32),
                     jax.ShapeDtypeStruct((D_H, D_H), jnp.float32)]
    else:
        Wc, bc = final
        body = _post_final_body
        in_specs += [pl.BlockSpec((D_H, D_OUT), lambda i: (0, 0)),
                     pl.BlockSpec((1, D_OUT), lambda i: (0, 0))]
        args += [Wc, bc.reshape(1, D_OUT)]
        out_spec = pl.BlockSpec((BLK, D_OUT), lambda i: (i, 0))
        out_shape = jax.ShapeDtypeStruct((NPAD, D_OUT), jnp.float32)
    return pl.pallas_call(
        body,
        grid=(GRID,),
        in_specs=in_specs,
        out_specs=out_spec,
        out_shape=out_shape,
    )(*args)


def kernel(x, edge_index, W1, b1, WV1, W2, b2, WV2, Wc, bc):
    src = edge_index[0].astype(jnp.int32)
    dst = edge_index[1].astype(jnp.int32)

    degp = _deg_partials(dst)
    mask = (jnp.arange(NPAD) < N).astype(jnp.float32)[:, None]

    h = jnp.zeros((NPAD, D_H), jnp.float32).at[:N].set(x)

    dinv, y, hn, hth = _layer_y1(degp, h, W1)
    Sp = _edge_scatter(y, src, dst)
    y, hn, hth = _layer_post(Sp, y, dinv, hn, hth, WV1, b1, mask, mid=W2)

    Sp = _edge_scatter(y, src, dst)
    outp = _layer_post(Sp, y, dinv, hn, hth, WV2, b2, mask, final=(Wc, bc))

    out = outp[:N]
    qel = jnp.array(0.0, dtype=jnp.float32)
    return (out, qel)


# speedup vs baseline: 1.1301x; 1.0083x over previous
import functools
import jax
import jax.numpy as jnp
from jax import lax
from jax.experimental import pallas as pl
from jax.experimental.pallas import tpu as pltpu
from jax.experimental.pallas import tpu_sc as plsc

N = 10000
E = 320000
D_H = 128
D_OUT = 64
BETA = 0.1

NC = 2
NS = 16
NW = NC * NS
CHUNK = 128
FULL = 78
EXTRA_BASE = NW * FULL * CHUNK
EPT = E // NW
NPAD = 10240
STRIPE = NPAD // NS

_sc_params = pltpu.CompilerParams(needs_layout_passes=False)


@functools.cache
def _sc_mesh():
    return plsc.VectorSubcoreMesh(core_axis_name="c", subcore_axis_name="s",
                                  num_cores=NC, num_subcores=NS)


def _deg_body(dst_hbm, out_hbm, dstv, hist):
    cid = lax.axis_index("c")
    sid = lax.axis_index("s")
    wid = sid * NC + cid

    z16 = jnp.zeros((16,), jnp.float32)

    def zero_row(i, _):
        hist[pl.ds(i * 16, 16)] = z16
        return 0

    lax.fori_loop(0, NPAD // 16, zero_row, 0)

    pltpu.sync_copy(dst_hbm.at[pl.ds(wid * EPT, EPT)], dstv)

    ones = jnp.full((16,), 1.0, jnp.float32)

    def count(i, _):
        d = dstv[pl.ds(i * 16, 16)]
        plsc.addupdate_scatter(hist, [d], ones)
        return 0

    lax.fori_loop(0, EPT // 16, count, 0)
    pltpu.sync_copy(hist, out_hbm.at[wid])


def _deg_partials(dst):
    return pl.kernel(
        _deg_body,
        out_type=jax.ShapeDtypeStruct((NW, NPAD), jnp.float32),
        mesh=_sc_mesh(),
        scratch_types=[
            pltpu.VMEM((EPT,), jnp.int32),
            pltpu.VMEM((NPAD,), jnp.float32),
        ],
        compiler_params=_sc_params,
    )(dst)


def _scatter_body(y_hbm, src_hbm, dst_hbm, out_hbm,
                  ssm, dsm, rows, isem, gsem, acc):
    cid = lax.axis_index("c")
    sid = lax.axis_index("s")
    wid = sid * NC + cid
    base = wid * (FULL * CHUNK)

    def idx_start(c, ib):
        off = base + c * CHUNK
        pltpu.async_copy(src_hbm.at[pl.ds(off, CHUNK)], ssm[ib], isem[ib])
        pltpu.async_copy(dst_hbm.at[pl.ds(off, CHUNK)], dsm[ib], isem[ib])

    def idx_wait(c, ib):
        off = base + c * CHUNK
        pltpu.make_async_copy(src_hbm.at[pl.ds(off, CHUNK)], ssm[ib],
                              isem[ib]).wait()
        pltpu.make_async_copy(dst_hbm.at[pl.ds(off, CHUNK)], dsm[ib],
                              isem[ib]).wait()

    def gather_start(ib, rb):
        pltpu.async_copy(y_hbm.at[ssm[ib]], rows[rb], gsem[rb])

    def gather_wait(ib, rb):
        pltpu.make_async_copy(y_hbm.at[ssm[ib]], rows[rb], gsem[rb]).wait()

    idx_start(0, 0)
    idx_start(1, 1)

    z16 = jnp.zeros((16,), jnp.float32)

    def zero_row(i, _):
        for t in range(D_H // 16):
            rows[0][i, pl.ds(t * 16, 16)] = z16
        return 0

    lax.fori_loop(0, CHUNK, zero_row, 0)
    for k in range(STRIPE // CHUNK):
        pltpu.sync_copy(rows[0], acc.at[pl.ds(sid * STRIPE + k * CHUNK,
                                              CHUNK)])

    idx_wait(0, 0)
    gather_start(0, 0)
    plsc.subcore_barrier()

    def body(j, _):
        c0 = 6 * j
        for b in range(6):
            c = c0 + b
            ib, ib1, ib2 = b % 3, (b + 1) % 3, (b + 2) % 3
            rb, rb1 = b % 2, (b + 1) % 2

            @pl.when(c + 2 < FULL)
            def _():
                idx_start(c + 2, ib2)

            @pl.when(c + 1 < FULL)
            def _():
                idx_wait(c + 1, ib1)
                gather_start(ib1, rb1)

            gather_wait(ib, rb)
            pltpu.sync_copy(rows[rb], acc.at[dsm[ib]], add=True)
        return 0

    lax.fori_loop(0, FULL // 6, body, 0)

    @pl.when(wid < 4)
    def _():
        ebase = EXTRA_BASE + wid * CHUNK
        pltpu.sync_copy(src_hbm.at[pl.ds(ebase, CHUNK)], ssm[0])
        pltpu.async_copy(y_hbm.at[ssm[0]], rows[0], gsem[0]).wait()
        pltpu.sync_copy(dst_hbm.at[pl.ds(ebase, CHUNK)], dsm[0])
        pltpu.sync_copy(rows[0], acc.at[dsm[0]], add=True)

    plsc.subcore_barrier()
    pltpu.sync_copy(acc.at[pl.ds(sid * STRIPE, STRIPE)],
                    out_hbm.at[cid, pl.ds(sid * STRIPE, STRIPE)])


def _edge_scatter(y, src, dst):
    return pl.kernel(
        _scatter_body,
        out_type=jax.ShapeDtypeStruct((NC, NPAD, D_H), jnp.float32),
        mesh=_sc_mesh(),
        scratch_types=[
            [pltpu.VMEM((CHUNK,), jnp.int32) for _ in range(3)],
            [pltpu.VMEM((CHUNK,), jnp.int32) for _ in range(3)],
            [pltpu.VMEM((CHUNK, D_H), jnp.float32) for _ in range(2)],
            [pltpu.SemaphoreType.DMA for _ in range(3)],
            [pltpu.SemaphoreType.DMA for _ in range(2)],
            pltpu.VMEM_SHARED((NPAD, D_H), jnp.float32),
        ],
        compiler_params=_sc_params,
    )(y, src, dst)


BLK = 2560
GRID = NPAD // BLK


def _norm_accum(h, hn_ref, hth_ref):
    nrm = jnp.sqrt(jnp.sum(h * h, axis=1, keepdims=True))
    hn = h / (nrm + 1e-12)
    hn_ref[...] = hn
    hth = lax.dot_general(hn, hn, (((0,), (0,)), ((), ())),
                          preferred_element_type=jnp.float32)

    @pl.when(pl.program_id(0) == 0)
    def _():
        hth_ref[...] = jnp.zeros_like(hth_ref)

    hth_ref[...] += hth


def _y1_body(degp_ref, h_ref, w_ref, dinv_ref, y_ref, hn_ref, hth_ref):
    deg = jnp.sum(degp_ref[...], axis=0, keepdims=True) + 1.0
    dinv = lax.rsqrt(deg).T
    dinv_ref[...] = dinv
    h = h_ref[...]
    y_ref[...] = jnp.dot(h, w_ref[...],
                         preferred_element_type=jnp.float32) * dinv
    _norm_accum(h, hn_ref, hth_ref)


def _layer_y1(degp, h, W):
    return pl.pallas_call(
        _y1_body,
        grid=(GRID,),
        in_specs=[
            pl.BlockSpec((NW, BLK), lambda i: (0, i)),
            pl.BlockSpec((BLK, D_H), lambda i: (i, 0)),
            pl.BlockSpec((D_H, D_H), lambda i: (0, 0)),
        ],
        out_specs=[
            pl.BlockSpec((BLK, 1), lambda i: (i, 0)),
            pl.BlockSpec((BLK, D_H), lambda i: (i, 0)),
            pl.BlockSpec((BLK, D_H), lambda i: (i, 0)),
            pl.BlockSpec((D_H, D_H), lambda i: (0, 0)),
        ],
        out_shape=[
            jax.ShapeDtypeStruct((NPAD, 1), jnp.float32),
            jax.ShapeDtypeStruct((NPAD, D_H), jnp.float32),
            jax.ShapeDtypeStruct((NPAD, D_H), jnp.float32),
            jax.ShapeDtypeStruct((D_H, D_H), jnp.float32),
        ],
    )(degp, h, W)


def _post_h(s0, s1, y, dinv, hn, hth, wv, b, mask):
    m = jnp.dot(hth[...], wv[...],
                preferred_element_type=jnp.float32) * (BETA / N)
    z = ((s0[...] + s1[...] + y[...]) * dinv[...]
         + jnp.dot(hn[...], m, preferred_element_type=jnp.float32)
         + b[...])
    z = jnp.maximum(z, 0.0)
    nrm = jnp.sqrt(jnp.sum(z * z, axis=1, keepdims=True))
    return (z / (nrm + 1e-12)) * mask[...]


def _post_mid_body(s0, s1, y, dinv, hn, hth, wv, b, mask, w2,
                   y2_ref, hn2_ref, hth2_ref):
    h = _post_h(s0, s1, y, dinv, hn, hth, wv, b, mask)
    y2_ref[...] = jnp.dot(h, w2[...],
                          preferred_element_type=jnp.float32) * dinv[...]
    _norm_accum(h, hn2_ref, hth2_ref)


def _post_final_body(s0, s1, y, dinv, hn, hth, wv, b, mask, wc, bc, o_ref):
    h = _post_h(s0, s1, y, dinv, hn, hth, wv, b, mask)
    o_ref[...] = jnp.dot(h, wc[...],
                         preferred_element_type=jnp.float32) + bc[...]


def _layer_post(Sp, y, dinv, hn, hth, WV, b, mask, final=None, mid=None):
    row = lambda: pl.BlockSpec((BLK, D_H), lambda i: (i, 0))
    sq = pl.BlockSpec((D_H, D_H), lambda i: (0, 0))
    col = pl.BlockSpec((BLK, 1), lambda i: (i, 0))
    in_specs = [row(), row(), row(), col, row(), sq, sq,
                pl.BlockSpec((1, D_H), lambda i: (0, 0)), col]
    args = [Sp[0], Sp[1], y, dinv, hn, hth, WV, b.reshape(1, D_H), mask]
    if mid is not None:
        body = _post_mid_body
        in_specs += [sq]
        args += [mid]
        out_spec = [row(), row(), sq]
        out_shape = [jax.ShapeDtypeStruct((NPAD, D_H), jnp.float32),
                     jax.ShapeDtypeStruct((NPAD, D_H), jnp.float32),
                     jax.ShapeDtypeStruct((D_H, D_H), jnp.float32)]
    else:
        Wc, bc = final
        body = _post_final_body
        in_specs += [pl.BlockSpec((D_H, D_OUT), lambda i: (0, 0)),
                     pl.BlockSpec((1, D_OUT), lambda i: (0, 0))]
        args += [Wc, bc.reshape(1, D_OUT)]
        out_spec = pl.BlockSpec((BLK, D_OUT), lambda i: (i, 0))
        out_shape = jax.ShapeDtypeStruct((NPAD, D_OUT), jnp.float32)
    return pl.pallas_call(
        body,
        grid=(GRID,),
        in_specs=in_specs,
        out_specs=out_spec,
        out_shape=out_shape,
    )(*args)


def kernel(x, edge_index, W1, b1, WV1, W2, b2, WV2, Wc, bc):
    src = edge_index[0].astype(jnp.int32)
    dst = edge_index[1].astype(jnp.int32)

    degp = _deg_partials(dst)
    mask = (jnp.arange(NPAD) < N).astype(jnp.float32)[:, None]

    h = jnp.zeros((NPAD, D_H), jnp.float32).at[:N].set(x)

    dinv, y, hn, hth = _layer_y1(degp, h, W1)
    Sp = _edge_scatter(y, src, dst)
    y, hn, hth = _layer_post(Sp, y, dinv, hn, hth, WV1, b1, mask, mid=W2)

    Sp = _edge_scatter(y, src, dst)
    outp = _layer_post(Sp, y, dinv, hn, hth, WV2, b2, mask, final=(Wc, bc))

    out = outp[:N]
    qel = jnp.array(0.0, dtype=jnp.float32)
    return (out, qel)
